# baseline (speedup 1.0000x reference)
import functools

import jax
import jax.numpy as jnp
from jax import lax
from jax.experimental import pallas as pl
from jax.experimental.pallas import tpu as pltpu
from jax.experimental.pallas import tpu_sc as plsc

B, L, D = 16, 2048, 128
NC, NS = 2, 16
NW = NC * NS
CW = L // NW
BPC = 2
CR = BPC * CW
NCHUNK = B // BPC
LANES = 16
NBUF = 6
AHEAD = 4

_mesh = plsc.VectorSubcoreMesh(core_axis_name="c", subcore_axis_name="s")


@functools.partial(
    pl.kernel,
    out_type=jax.ShapeDtypeStruct((B, L, D), jnp.float32),
    mesh=_mesh,
    scratch_types=[
        pltpu.VMEM((NCHUNK, CR), jnp.int32),
        pltpu.VMEM((CW, D), jnp.float32),
        pltpu.VMEM((NBUF, CR, D), jnp.float32),
        pltpu.SemaphoreType.DMA((NBUF,)),
        pltpu.SemaphoreType.DMA((NBUF,)),
        pltpu.SemaphoreType.DMA,
    ],
)
def _tok_pos(x_hbm, txt_hbm, pos_hbm, out_hbm, idx_v, pos_v, rows_v,
             gsem, osem, psem):
    wid = lax.axis_index("s") * NC + lax.axis_index("c")
    col = wid * CW
    pltpu.sync_copy(x_hbm.at[wid], idx_v)

    def gather(j):
        bb = j % NBUF
        return pltpu.async_copy(
            txt_hbm.at[idx_v.at[j]], rows_v.at[bb], gsem.at[bb])

    def writeback_half(j, p):
        bb = j % NBUF
        return pltpu.async_copy(
            rows_v.at[bb, pl.ds(p * CW, CW)],
            out_hbm.at[BPC * j + p, pl.ds(col, CW)], osem.at[bb])

    def add_pos(bb):
        rv = rows_v.at[bb]

        def row_body(r, carry):
            for p in range(BPC):
                for t in range(D // LANES):
                    sl = pl.ds(t * LANES, LANES)
                    rv[p * CW + r, sl] = rv[p * CW + r, sl] + pos_v[r, sl]
            return carry

        lax.fori_loop(0, CW, row_body, 0)

    gat, out = {}, {}
    for j in range(AHEAD):
        gat[j] = gather(j)
    pltpu.async_copy(pos_hbm.at[pl.ds(col, CW)], pos_v, psem).wait()
    for j in range(NCHUNK):
        jn = j + AHEAD
        if jn < NCHUNK:
            if jn - NBUF >= 0:
                out[jn - NBUF][0].wait()
                out[jn - NBUF][1].wait()
            gat[jn] = gather(jn)
        gat[j].wait()
        add_pos(j % NBUF)
        out[j] = (writeback_half(j, 0), writeback_half(j, 1))
    for j in range(NCHUNK - NBUF, NCHUNK):
        out[j][0].wait()
        out[j][1].wait()


def kernel(x, txt_emb, pos_emb):
    xr = (x.reshape(NCHUNK, BPC, NW, CW)
          .transpose(2, 0, 1, 3)
          .reshape(NW, NCHUNK, CR)
          .astype(jnp.int32))
    return _tok_pos(xr, txt_emb, pos_emb)


# speedup vs baseline: 1.0038x; 1.0038x over previous
import functools

import jax
import jax.numpy as jnp
from jax import lax
from jax.experimental import pallas as pl
from jax.experimental.pallas import tpu as pltpu
from jax.experimental.pallas import tpu_sc as plsc

B, L, D = 16, 2048, 128
NC, NS = 2, 16
NW = NC * NS
CW = L // NW
BPC = 2
CR = BPC * CW
NCHUNK = B // BPC
LANES = 16
NBUF = 6
AHEAD = 4

_mesh = plsc.VectorSubcoreMesh(core_axis_name="c", subcore_axis_name="s")


@functools.partial(
    pl.kernel,
    out_type=jax.ShapeDtypeStruct((B, L, D), jnp.float32),
    mesh=_mesh,
    scratch_types=[
        pltpu.VMEM((NCHUNK, CR), jnp.int32),
        pltpu.VMEM((CR, D), jnp.float32),
        pltpu.VMEM((NBUF, CR, D), jnp.float32),
        pltpu.SemaphoreType.DMA((NBUF,)),
        pltpu.SemaphoreType.DMA((NBUF,)),
        pltpu.SemaphoreType.DMA,
    ],
)
def _tok_pos(x_hbm, txt_hbm, pos_hbm, out_hbm, idx_v, pos_v, rows_v,
             gsem, osem, psem):
    wid = lax.axis_index("s") * NC + lax.axis_index("c")
    col = wid * CW
    pltpu.sync_copy(x_hbm.at[wid], idx_v)

    def gather(j):
        bb = j % NBUF
        return pltpu.async_copy(
            txt_hbm.at[idx_v.at[j]], rows_v.at[bb], gsem.at[bb])

    def writeback(j):
        bb = j % NBUF
        d0 = pltpu.async_copy(
            rows_v.at[bb, pl.ds(0, CW)],
            out_hbm.at[BPC * j, pl.ds(col, CW)], osem.at[bb])
        d1 = pltpu.async_copy(
            rows_v.at[bb, pl.ds(CW, CW)],
            out_hbm.at[BPC * j + 1, pl.ds(col, CW)], osem.at[bb])
        return (d0, d1)

    def add_pos(bb):
        rv = rows_v.at[bb]

        def row_body(r, carry):
            for t in range(D // LANES):
                sl = pl.ds(t * LANES, LANES)
                rv[r, sl] = rv[r, sl] + pos_v[r, sl]
            return carry

        lax.fori_loop(0, CR, row_body, 0)

    gat, out = {}, {}
    for j in range(AHEAD):
        gat[j] = gather(j)
    pdesc = [
        pltpu.async_copy(pos_hbm.at[pl.ds(col, CW)],
                         pos_v.at[pl.ds(p * CW, CW)], psem)
        for p in range(BPC)
    ]
    for d in pdesc:
        d.wait()
    for j in range(NCHUNK):
        jn = j + AHEAD
        if jn < NCHUNK:
            if jn - NBUF >= 0:
                out[jn - NBUF][0].wait()
                out[jn - NBUF][1].wait()
            gat[jn] = gather(jn)
        gat[j].wait()
        add_pos(j % NBUF)
        out[j] = writeback(j)
    for j in range(NCHUNK - NBUF, NCHUNK):
        out[j][0].wait()
        out[j][1].wait()


def kernel(x, txt_emb, pos_emb):
    xr = (x.reshape(NCHUNK, BPC, NW, CW)
          .transpose(2, 0, 1, 3)
          .reshape(NW, NCHUNK, CR)
          .astype(jnp.int32))
    return _tok_pos(xr, txt_emb, pos_emb)
